# Initial kernel scaffold; baseline (speedup 1.0000x reference)
import jax
import jax.numpy as jnp
from jax.experimental import pallas as pl


def kernel(x, weight, bias, so_cw, so_cb, so_g, so_b, so_ew, so_eb, si_cw, si_cb, si_g, si_b, si_ew, si_eb):
    raise NotImplementedError("write your pallas kernel here")


# speedup vs baseline: 1.2169x; 1.2169x over previous
import functools

import jax
import jax.numpy as jnp
from jax import lax
from jax.experimental import pallas as pl
from jax.experimental.pallas import tpu as pltpu
from jax.experimental.pallas import tpu_sc as plsc

N = 2048
C1 = 4096
C2 = 4096
G = 1024
K = 8
KPAD = 16
EPS = 1e-5
BLK = 256
NBLK = N // BLK
BLK1 = 128
NBLK1 = N // BLK1

NW = 32
TPW = N // NW


def _comp_body(x0, x1, x2, x3, wo, wi, cbo, cbi, co_ref, ci_ref):
    a0, a1, a2, a3 = x0[...], x1[...], x2[...], x3[...]

    def grouped(w_ref, cb_ref):
        p0 = a0 * w_ref[0:1, :]
        p1 = a1 * w_ref[1:2, :]
        p2 = a2 * w_ref[2:3, :]
        p3 = a3 * w_ref[3:4, :]
        return (p0 + p2) + (p1 + p3) + cb_ref[...]

    co_ref[...] = grouped(wo, cbo)
    ci_ref[...] = grouped(wi, cbi)


def _compute_comp(xcols, cwo, cwi, cbo, cbi):
    blk = lambda: pl.BlockSpec((BLK1, G), lambda i: (i, 0))
    vec = lambda r: pl.BlockSpec((r, G), lambda i: (0, 0))
    return pl.pallas_call(
        _comp_body,
        grid=(NBLK1,),
        in_specs=[blk(), blk(), blk(), blk(), vec(4), vec(4), vec(1), vec(1)],
        out_specs=[blk(), blk()],
        out_shape=[
            jax.ShapeDtypeStruct((N, G), jnp.float32),
            jax.ShapeDtypeStruct((N, G), jnp.float32),
        ],
    )(*xcols, cwo, cwi, cbo, cbi)


def _stats_body(co_ref, ci_ref, og_ref, ob_ref, ig_ref, ib_ref,
                no_ref, ni_ref):
    def bn(c_ref, g_ref, b_ref, n_ref):
        c = c_ref[...]
        m = jnp.mean(c, axis=0, keepdims=True)
        d = c - m
        v = jnp.mean(d * d, axis=0, keepdims=True)
        n_ref[...] = (c - m) / jnp.sqrt(v + EPS) * g_ref[...] + b_ref[...]

    bn(co_ref, og_ref, ob_ref, no_ref)
    bn(ci_ref, ig_ref, ib_ref, ni_ref)


def _bn_normalize(co, ci, og, ob, ig, ib):
    return pl.pallas_call(
        _stats_body,
        out_shape=[
            jax.ShapeDtypeStruct((N, G), jnp.float32),
            jax.ShapeDtypeStruct((N, G), jnp.float32),
        ],
    )(co, ci, og, ob, ig, ib)


def _branch_topk(score, x, m0, z, want_xsel):
    iota = lax.broadcasted_iota(jnp.int32, score.shape, 1)
    lane = lax.broadcasted_iota(jnp.int32, (score.shape[0], KPAD), 1)
    idx_out = jnp.zeros((score.shape[0], KPAD), jnp.int32)
    sv_out = jnp.zeros((score.shape[0], KPAD), jnp.float32)
    s = score
    for k in range(K):
        m = jnp.max(s, axis=1, keepdims=True)
        am = jnp.min(jnp.where(s == m, iota, C2), axis=1, keepdims=True)
        onehot = iota == am
        sv = jnp.exp(m - m0) / z
        if want_xsel:
            xsel = jnp.sum(jnp.where(onehot, x, 0.0), axis=1, keepdims=True)
            sv = sv * xsel
        idx_out = jnp.where(lane == k, am, idx_out)
        sv_out = jnp.where(lane == k, sv, sv_out)
        s = jnp.where(onehot, -jnp.inf, s)
    return idx_out, sv_out


def _score_topk_body(want_xsel, act_ref, x_ref, ew_ref, eb_ref, i_ref, v_ref):
    score = jnp.dot(act_ref[...], ew_ref[...],
                    preferred_element_type=jnp.float32) + eb_ref[...]
    m0 = jnp.max(score, axis=1, keepdims=True)
    z = jnp.sum(jnp.exp(score - m0), axis=1, keepdims=True)
    i_out, v_out = _branch_topk(score, x_ref[...], m0, z, want_xsel)
    i_ref[...] = i_out
    v_ref[...] = v_out


def _score_topk(want_xsel, act, x2d, ew_t, eb):
    return pl.pallas_call(
        functools.partial(_score_topk_body, want_xsel),
        grid=(NBLK,),
        in_specs=[
            pl.BlockSpec((BLK, G), lambda i: (i, 0)),
            pl.BlockSpec((BLK, C1), lambda i: (i, 0)),
            pl.BlockSpec((G, C2), lambda i: (0, 0)),
            pl.BlockSpec((1, C2), lambda i: (0, 0)),
        ],
        out_specs=[pl.BlockSpec((BLK, KPAD), lambda i: (i, 0))] * 2,
        out_shape=[
            jax.ShapeDtypeStruct((N, KPAD), jnp.int32),
            jax.ShapeDtypeStruct((N, KPAD), jnp.float32),
        ],
    )(act, x2d, ew_t, eb)


def _sc_body(w_hbm, wt_hbm, x_hbm, bias_hbm, ii_hbm, gi_hbm, io_hbm, vo_hbm,
             out_hbm,
             ii_v, gi_v, io_v, vo_v, wrows, wtrows, xrow, orow, biasv, dscr,
             semw, semwt, semx):
    cid = lax.axis_index("c")
    sid = lax.axis_index("s")
    wid = sid * 2 + cid
    base = wid * TPW

    pltpu.sync_copy(bias_hbm.at[pl.ds(0, 1)], biasv)
    pltpu.sync_copy(ii_hbm.at[pl.ds(base, TPW)], ii_v)
    pltpu.sync_copy(gi_hbm.at[pl.ds(base, TPW)], gi_v)
    pltpu.sync_copy(io_hbm.at[pl.ds(base, TPW)], io_v)
    pltpu.sync_copy(vo_hbm.at[pl.ds(base, TPW)], vo_v)

    zeros16 = jnp.zeros((16,), jnp.float32)

    def token(j, _):
        t = base + j
        cw = pltpu.async_copy(w_hbm.at[ii_v.at[j, pl.ds(0, K)]], wrows, semw)
        cwt = pltpu.async_copy(wt_hbm.at[io_v.at[j, pl.ds(0, K)]], wtrows, semwt)
        cx = pltpu.async_copy(x_hbm.at[pl.ds(t, 1)], xrow, semx)
        cw.wait()
        cwt.wait()
        cx.wait()

        gv = gi_v[j, :]
        gvecs = [jnp.full((16,), gv[k], jnp.float32) for k in range(K)]

        def chunk(i, daccs):
            ds = pl.ds(i * 16, 16)
            xc = xrow[0, ds]
            acc = biasv[0, ds]
            new = []
            for k in range(K):
                acc = acc + gvecs[k] * wrows[k, ds]
                new.append(daccs[k] + xc * wtrows[k, ds])
            orow[0, ds] = acc
            return tuple(new)

        daccs = lax.fori_loop(0, C2 // 16, chunk, tuple(zeros16 for _ in range(K)))

        for k in range(K):
            dscr[k, :] = daccs[k]
        lane = lax.broadcasted_iota(jnp.int32, (16,), 0)
        rowidx = lane & 7
        dvec = zeros16
        for c in range(16):
            dvec = dvec + plsc.load_gather(dscr, [rowidx, jnp.full((16,), c, jnp.int32)])
        dvec = dvec * vo_v[j, :]

        plsc.addupdate_scatter(orow.at[0], [io_v[j, :]], dvec)
        pltpu.sync_copy(orow, out_hbm.at[pl.ds(t, 1)])
        return 0

    lax.fori_loop(0, TPW, token, 0)


def _sc_apply(w, wt, x2d, bias2d, ii, gi, io, vo):
    mesh = plsc.VectorSubcoreMesh(core_axis_name="c", subcore_axis_name="s")
    fn = pl.kernel(
        _sc_body,
        out_type=jax.ShapeDtypeStruct((N, C2), jnp.float32),
        mesh=mesh,
        compiler_params=pltpu.CompilerParams(needs_layout_passes=False),
        scratch_types=[
            pltpu.VMEM((TPW, KPAD), jnp.int32),
            pltpu.VMEM((TPW, KPAD), jnp.float32),
            pltpu.VMEM((TPW, KPAD), jnp.int32),
            pltpu.VMEM((TPW, KPAD), jnp.float32),
            pltpu.VMEM((K, C2), jnp.float32),
            pltpu.VMEM((K, C1), jnp.float32),
            pltpu.VMEM((1, C1), jnp.float32),
            pltpu.VMEM((1, C2), jnp.float32),
            pltpu.VMEM((1, C2), jnp.float32),
            pltpu.VMEM((K, 16), jnp.float32),
            pltpu.SemaphoreType.DMA,
            pltpu.SemaphoreType.DMA,
            pltpu.SemaphoreType.DMA,
        ],
    )
    return fn(w, wt, x2d, bias2d, ii, gi, io, vo)


def kernel(x, weight, bias, so_cw, so_cb, so_g, so_b, so_ew, so_eb,
           si_cw, si_cb, si_g, si_b, si_ew, si_eb):
    b, l, _ = x.shape
    x2d = x.reshape(b * l, C1)

    wt = weight.T

    def scores(cw, cb, gamma, beta, ew, eb):
        grp = cw.shape[0]
        xg = x2d.reshape(b * l, grp, 4)
        comp = jnp.einsum('ngj,gj->ng', xg, cw) + cb
        mean = jnp.mean(comp, axis=0)
        var = jnp.var(comp, axis=0)
        normed = (comp - mean) / jnp.sqrt(var + EPS) * gamma + beta
        act = jax.nn.gelu(normed, approximate=False)
        return act @ ew.T + eb

    sn_o = jax.nn.softmax(scores(so_cw, so_cb, so_g, so_b, so_ew, so_eb)
                          .reshape(b, l, C2), axis=-1)
    s_v_out, s_i_out = lax.top_k(sn_o, K)
    sn_i = jax.nn.softmax(scores(si_cw, si_cb, si_g, si_b, si_ew, si_eb)
                          .reshape(b, l, C1), axis=-1)
    s_v_in, s_i_in = lax.top_k(sn_i, K)
    x_gated = jnp.take_along_axis(x, s_i_in, axis=-1) * s_v_in

    pad_i = jnp.zeros((N, KPAD - K), jnp.int32)
    pad_f = jnp.zeros((N, KPAD - K), jnp.float32)
    io = jnp.concatenate([s_i_out.reshape(N, K), pad_i], axis=1)
    vo = jnp.concatenate([s_v_out.reshape(N, K), pad_f], axis=1)
    ii = jnp.concatenate([s_i_in.reshape(N, K), pad_i], axis=1)
    vi = jnp.concatenate([x_gated.reshape(N, K), pad_f], axis=1)

    out = _sc_apply(weight, wt, x2d, bias.reshape(1, C2), ii, vi, io, vo)
    return out.reshape(b, l, C2)


# speedup vs baseline: 6.1710x; 5.0710x over previous
import jax
import jax.numpy as jnp
from jax import lax
from jax.experimental import pallas as pl
from jax.experimental.pallas import tpu as pltpu
from jax.experimental.pallas import tpu_sc as plsc

N = 2048
C1 = 4096
C2 = 4096
G = 1024
K = 8
KPAD = 16
EPS = 1e-5
BLK = 256
NBLK = N // BLK

NW = 32
TPW = N // NW


def _branch_topk(score, x, m0, z, want_xsel):
    iota = lax.broadcasted_iota(jnp.int32, score.shape, 1)
    lane = lax.broadcasted_iota(jnp.int32, (score.shape[0], KPAD), 1)
    idx_out = jnp.zeros((score.shape[0], KPAD), jnp.int32)
    sv_out = jnp.zeros((score.shape[0], KPAD), jnp.float32)
    s = score
    for k in range(K):
        m = jnp.max(s, axis=1, keepdims=True)
        am = jnp.min(jnp.where(s == m, iota, C2), axis=1, keepdims=True)
        onehot = iota == am
        sv = jnp.exp(m - m0) / z
        if want_xsel:
            xsel = jnp.sum(jnp.where(onehot, x, 0.0), axis=1, keepdims=True)
            sv = sv * xsel
        idx_out = jnp.where(lane == k, am, idx_out)
        sv_out = jnp.where(lane == k, sv, sv_out)
        s = jnp.where(onehot, -jnp.inf, s)
    return idx_out, sv_out


def _topk_body(so_ref, si_ref, x_ref, io_ref, vo_ref, ii_ref, vi_ref):
    def branch(score, x, want_xsel, i_ref, v_ref):
        m0 = jnp.max(score, axis=1, keepdims=True)
        z = jnp.sum(jnp.exp(score - m0), axis=1, keepdims=True)
        i_out, v_out = _branch_topk(score, x, m0, z, want_xsel)
        i_ref[...] = i_out
        v_ref[...] = v_out

    branch(so_ref[...], None, False, io_ref, vo_ref)
    branch(si_ref[...], x_ref[...], True, ii_ref, vi_ref)


def _topk(so, si, x2d):
    return pl.pallas_call(
        _topk_body,
        grid=(NBLK,),
        in_specs=[
            pl.BlockSpec((BLK, C2), lambda i: (i, 0)),
            pl.BlockSpec((BLK, C1), lambda i: (i, 0)),
            pl.BlockSpec((BLK, C1), lambda i: (i, 0)),
        ],
        out_specs=[pl.BlockSpec((BLK, KPAD), lambda i: (i, 0))] * 4,
        out_shape=[
            jax.ShapeDtypeStruct((N, KPAD), jnp.int32),
            jax.ShapeDtypeStruct((N, KPAD), jnp.float32),
            jax.ShapeDtypeStruct((N, KPAD), jnp.int32),
            jax.ShapeDtypeStruct((N, KPAD), jnp.float32),
        ],
    )(so, si, x2d)


def _sc_body(w_hbm, wt_hbm, x_hbm, bias_hbm, ii_hbm, gi_hbm, io_hbm, vo_hbm,
             out_hbm,
             ii_v, gi_v, io_v, vo_v, wrows, wtrows, xrow, orow, biasv, dscr,
             semw, semwt, semx):
    cid = lax.axis_index("c")
    sid = lax.axis_index("s")
    wid = sid * 2 + cid
    base = wid * TPW

    pltpu.sync_copy(bias_hbm.at[pl.ds(0, 1)], biasv)
    pltpu.sync_copy(ii_hbm.at[pl.ds(base, TPW)], ii_v)
    pltpu.sync_copy(gi_hbm.at[pl.ds(base, TPW)], gi_v)
    pltpu.sync_copy(io_hbm.at[pl.ds(base, TPW)], io_v)
    pltpu.sync_copy(vo_hbm.at[pl.ds(base, TPW)], vo_v)

    zeros16 = jnp.zeros((16,), jnp.float32)

    def token(j, _):
        t = base + j
        cw = pltpu.async_copy(w_hbm.at[ii_v.at[j, pl.ds(0, K)]], wrows, semw)
        cwt = pltpu.async_copy(wt_hbm.at[io_v.at[j, pl.ds(0, K)]], wtrows, semwt)
        cx = pltpu.async_copy(x_hbm.at[pl.ds(t, 1)], xrow, semx)
        cw.wait()
        cwt.wait()
        cx.wait()

        gv = gi_v[j, :]
        gvecs = [jnp.full((16,), gv[k], jnp.float32) for k in range(K)]

        def chunk(i, daccs):
            ds = pl.ds(i * 16, 16)
            xc = xrow[0, ds]
            acc = biasv[0, ds]
            new = []
            for k in range(K):
                acc = acc + gvecs[k] * wrows[k, ds]
                new.append(daccs[k] + xc * wtrows[k, ds])
            orow[0, ds] = acc
            return tuple(new)

        daccs = lax.fori_loop(0, C2 // 16, chunk, tuple(zeros16 for _ in range(K)))

        for k in range(K):
            dscr[k, :] = daccs[k]
        lane = lax.broadcasted_iota(jnp.int32, (16,), 0)
        rowidx = lane & 7
        dvec = zeros16
        for c in range(16):
            dvec = dvec + plsc.load_gather(dscr, [rowidx, jnp.full((16,), c, jnp.int32)])
        dvec = dvec * vo_v[j, :]

        plsc.addupdate_scatter(orow.at[0], [io_v[j, :]], dvec)
        pltpu.sync_copy(orow, out_hbm.at[pl.ds(t, 1)])
        return 0

    lax.fori_loop(0, TPW, token, 0)


def _sc_apply(w, wt, x2d, bias2d, ii, gi, io, vo):
    mesh = plsc.VectorSubcoreMesh(core_axis_name="c", subcore_axis_name="s")
    fn = pl.kernel(
        _sc_body,
        out_type=jax.ShapeDtypeStruct((N, C2), jnp.float32),
        mesh=mesh,
        compiler_params=pltpu.CompilerParams(needs_layout_passes=False),
        scratch_types=[
            pltpu.VMEM((TPW, KPAD), jnp.int32),
            pltpu.VMEM((TPW, KPAD), jnp.float32),
            pltpu.VMEM((TPW, KPAD), jnp.int32),
            pltpu.VMEM((TPW, KPAD), jnp.float32),
            pltpu.VMEM((K, C2), jnp.float32),
            pltpu.VMEM((K, C1), jnp.float32),
            pltpu.VMEM((1, C1), jnp.float32),
            pltpu.VMEM((1, C2), jnp.float32),
            pltpu.VMEM((1, C2), jnp.float32),
            pltpu.VMEM((K, 16), jnp.float32),
            pltpu.SemaphoreType.DMA,
            pltpu.SemaphoreType.DMA,
            pltpu.SemaphoreType.DMA,
        ],
    )
    return fn(w, wt, x2d, bias2d, ii, gi, io, vo)


def kernel(x, weight, bias, so_cw, so_cb, so_g, so_b, so_ew, so_eb,
           si_cw, si_cb, si_g, si_b, si_ew, si_eb):
    b, l, _ = x.shape
    x2d = x.reshape(b * l, C1)

    wt = weight.T

    def scores(cw, cb, gamma, beta, ew, eb):
        grp = cw.shape[0]
        xg = x2d.reshape(b * l, grp, 4)
        comp = jnp.einsum('ngj,gj->ng', xg, cw) + cb
        mean = jnp.mean(comp, axis=0)
        var = jnp.var(comp, axis=0)
        normed = (comp - mean) / jnp.sqrt(var + EPS) * gamma + beta
        act = jax.nn.gelu(normed, approximate=False)
        return act @ ew.T + eb

    so = scores(so_cw, so_cb, so_g, so_b, so_ew, so_eb)
    si = scores(si_cw, si_cb, si_g, si_b, si_ew, si_eb)
    io, vo, ii, vi = _topk(so, si, x2d)

    out = _sc_apply(weight, wt, x2d, bias.reshape(1, C2), ii, vi, io, vo)
    return out.reshape(b, l, C2)


# speedup vs baseline: 8.0094x; 1.2979x over previous
import jax
import jax.numpy as jnp
from jax import lax
from jax.experimental import pallas as pl
from jax.experimental.pallas import tpu as pltpu
from jax.experimental.pallas import tpu_sc as plsc

N = 2048
C1 = 4096
C2 = 4096
G = 1024
K = 8
KPAD = 16
EPS = 1e-5
BLK = 256
NBLK = N // BLK

NW = 32
TPW = N // NW


def _branch_topk(score, x, m0, z, want_xsel):
    iota = lax.broadcasted_iota(jnp.int32, score.shape, 1)
    lane = lax.broadcasted_iota(jnp.int32, (score.shape[0], KPAD), 1)
    idx_out = jnp.zeros((score.shape[0], KPAD), jnp.int32)
    sv_out = jnp.zeros((score.shape[0], KPAD), jnp.float32)
    s = score
    for k in range(K):
        m = jnp.max(s, axis=1, keepdims=True)
        am = jnp.min(jnp.where(s == m, iota, C2), axis=1, keepdims=True)
        onehot = iota == am
        sv = jnp.exp(m - m0) / z
        if want_xsel:
            xsel = jnp.sum(jnp.where(onehot, x, 0.0), axis=1, keepdims=True)
            sv = sv * xsel
        idx_out = jnp.where(lane == k, am, idx_out)
        sv_out = jnp.where(lane == k, sv, sv_out)
        s = jnp.where(onehot, -jnp.inf, s)
    return idx_out, sv_out


def _topk_body(so_ref, si_ref, x_ref, io_ref, vo_ref, ii_ref, vi_ref):
    def branch(score, x, want_xsel, i_ref, v_ref):
        m0 = jnp.max(score, axis=1, keepdims=True)
        z = jnp.sum(jnp.exp(score - m0), axis=1, keepdims=True)
        i_out, v_out = _branch_topk(score, x, m0, z, want_xsel)
        i_ref[...] = i_out
        v_ref[...] = v_out

    branch(so_ref[...], None, False, io_ref, vo_ref)
    branch(si_ref[...], x_ref[...], True, ii_ref, vi_ref)


def _topk(so, si, x2d):
    return pl.pallas_call(
        _topk_body,
        grid=(NBLK,),
        in_specs=[
            pl.BlockSpec((BLK, C2), lambda i: (i, 0)),
            pl.BlockSpec((BLK, C1), lambda i: (i, 0)),
            pl.BlockSpec((BLK, C1), lambda i: (i, 0)),
        ],
        out_specs=[pl.BlockSpec((BLK, KPAD), lambda i: (i, 0))] * 4,
        out_shape=[
            jax.ShapeDtypeStruct((N, KPAD), jnp.int32),
            jax.ShapeDtypeStruct((N, KPAD), jnp.float32),
            jax.ShapeDtypeStruct((N, KPAD), jnp.int32),
            jax.ShapeDtypeStruct((N, KPAD), jnp.float32),
        ],
    )(so, si, x2d)


H = C2 // 2
HCH = H // 16


def _sc_body(w_hbm, wt_hbm, x_hbm, bias_hbm, ii_hbm, gi_hbm, io_hbm, vo_hbm,
             out_hbm,
             ii_v, gi_v, io_v, vo_v, wrows, wtrows, xh, orow, biasv, dscr,
             semw0, semw1, semwt0, semwt1, semx0, semx1, semo0, semo1):
    cid = lax.axis_index("c")
    sid = lax.axis_index("s")
    wid = sid * 2 + cid
    base = wid * TPW
    semw = (semw0, semw1)
    semwt = (semwt0, semwt1)
    semx = (semx0, semx1)
    semo = (semo0, semo1)

    pltpu.sync_copy(bias_hbm.at[pl.ds(0, 1)], biasv)
    pltpu.sync_copy(ii_hbm.at[pl.ds(base, TPW)], ii_v)
    pltpu.sync_copy(gi_hbm.at[pl.ds(base, TPW)], gi_v)
    pltpu.sync_copy(io_hbm.at[pl.ds(base, TPW)], io_v)
    pltpu.sync_copy(vo_hbm.at[pl.ds(base, TPW)], vo_v)

    zeros16 = jnp.zeros((16,), jnp.float32)

    def fire(j, h, p):
        pltpu.async_copy(
            w_hbm.at[ii_v.at[j, pl.ds(0, K)], pl.ds(h * H, H)],
            wrows.at[p], semw[p])
        pltpu.async_copy(
            wt_hbm.at[io_v.at[j, pl.ds(0, K)], pl.ds(h * H, H)],
            wtrows.at[p], semwt[p])
        pltpu.async_copy(
            x_hbm.at[pl.ds(base + j, 1), pl.ds(h * H, H)],
            xh.at[p], semx[p])

    def wait_in(p):
        pltpu.make_async_copy(w_hbm.at[pl.ds(0, K), pl.ds(0, H)],
                              wrows.at[p], semw[p]).wait()
        pltpu.make_async_copy(wt_hbm.at[pl.ds(0, K), pl.ds(0, H)],
                              wtrows.at[p], semwt[p]).wait()
        pltpu.make_async_copy(x_hbm.at[pl.ds(0, 1), pl.ds(0, H)],
                              xh.at[p], semx[p]).wait()

    fire(0, 0, 0)

    def token(j, op):
        t = base + j

        @pl.when(j >= 2)
        def _():
            pltpu.make_async_copy(orow.at[op], out_hbm.at[pl.ds(0, 1)],
                                  semo[op]).wait()

        gv = gi_v[j, :]
        gvecs = [jnp.full((16,), gv[k], jnp.float32) for k in range(K)]

        daccs = tuple(zeros16 for _ in range(K))
        for h in (0, 1):
            p = h
            wait_in(p)
            if h == 0:
                fire(j, 1, 1)
            else:
                @pl.when(j + 1 < TPW)
                def _():
                    fire(j + 1, 0, 0)

            def chunk(i, dc, h=h, p=p):
                ds16 = pl.ds(i * 16, 16)
                dsfull = pl.ds(h * H + i * 16, 16)
                xc = xh[p, 0, ds16]
                acc = biasv[0, dsfull]
                new = []
                for k in range(K):
                    acc = acc + gvecs[k] * wrows[p, k, ds16]
                    new.append(dc[k] + xc * wtrows[p, k, ds16])
                orow[op, 0, dsfull] = acc
                return tuple(new)

            daccs = lax.fori_loop(0, HCH, chunk, daccs)

        for k in range(K):
            dscr[k, :] = daccs[k]
        lane = lax.broadcasted_iota(jnp.int32, (16,), 0)
        rowidx = lane & 7
        dvec = zeros16
        for c in range(16):
            dvec = dvec + plsc.load_gather(dscr, [rowidx, jnp.full((16,), c, jnp.int32)])
        dvec = dvec * vo_v[j, :]

        plsc.addupdate_scatter(orow.at[op, 0], [io_v[j, :]], dvec)
        pltpu.async_copy(orow.at[op], out_hbm.at[pl.ds(t, 1)], semo[op])

    def token_pair(i, _):
        token(2 * i, 0)
        token(2 * i + 1, 1)
        return 0

    lax.fori_loop(0, TPW // 2, token_pair, 0)

    for pp in (0, 1):
        pltpu.make_async_copy(orow.at[pp], out_hbm.at[pl.ds(0, 1)],
                              semo[pp]).wait()


def _sc_apply(w, wt, x2d, bias2d, ii, gi, io, vo):
    mesh = plsc.VectorSubcoreMesh(core_axis_name="c", subcore_axis_name="s")
    fn = pl.kernel(
        _sc_body,
        out_type=jax.ShapeDtypeStruct((N, C2), jnp.float32),
        mesh=mesh,
        compiler_params=pltpu.CompilerParams(needs_layout_passes=False),
        scratch_types=[
            pltpu.VMEM((TPW, KPAD), jnp.int32),
            pltpu.VMEM((TPW, KPAD), jnp.float32),
            pltpu.VMEM((TPW, KPAD), jnp.int32),
            pltpu.VMEM((TPW, KPAD), jnp.float32),
            pltpu.VMEM((2, K, H), jnp.float32),
            pltpu.VMEM((2, K, H), jnp.float32),
            pltpu.VMEM((2, 1, H), jnp.float32),
            pltpu.VMEM((2, 1, C2), jnp.float32),
            pltpu.VMEM((1, C2), jnp.float32),
            pltpu.VMEM((K, 16), jnp.float32),
        ] + [pltpu.SemaphoreType.DMA] * 8,
    )
    return fn(w, wt, x2d, bias2d, ii, gi, io, vo)


def kernel(x, weight, bias, so_cw, so_cb, so_g, so_b, so_ew, so_eb,
           si_cw, si_cb, si_g, si_b, si_ew, si_eb):
    b, l, _ = x.shape
    x2d = x.reshape(b * l, C1)

    wt = weight.T

    def scores(cw, cb, gamma, beta, ew, eb):
        grp = cw.shape[0]
        xg = x2d.reshape(b * l, grp, 4)
        comp = jnp.einsum('ngj,gj->ng', xg, cw) + cb
        mean = jnp.mean(comp, axis=0)
        var = jnp.var(comp, axis=0)
        normed = (comp - mean) / jnp.sqrt(var + EPS) * gamma + beta
        act = jax.nn.gelu(normed, approximate=False)
        return act @ ew.T + eb

    so = scores(so_cw, so_cb, so_g, so_b, so_ew, so_eb)
    si = scores(si_cw, si_cb, si_g, si_b, si_ew, si_eb)
    io, vo, ii, vi = _topk(so, si, x2d)

    out = _sc_apply(weight, wt, x2d, bias.reshape(1, C2), ii, vi, io, vo)
    return out.reshape(b, l, C2)
